# Initial kernel scaffold; baseline (speedup 1.0000x reference)
import jax
import jax.numpy as jnp
from jax.experimental import pallas as pl


def kernel(input_tensor):
    raise NotImplementedError("write your pallas kernel here")


# speedup vs baseline: 5.2329x; 5.2329x over previous
import functools

import jax
import jax.numpy as jnp
from jax.experimental import pallas as pl

_DEP_DIM = 4
_SLOT_NUM = 4096
_HASH_SEED = 1
_BLOCK_B = 256


def _onehot_kernel(out_ref, *, batch_size, block_b):
    d = pl.program_id(0)
    ib = pl.program_id(1)
    b = jax.lax.broadcasted_iota(jnp.int32, (block_b, 1), 0) + ib * block_b
    k = b * _DEP_DIM + d
    m = (k % batch_size) * _DEP_DIM + (k // batch_size)
    h = m.astype(jnp.uint32) * jnp.uint32(2654435761) + jnp.uint32(_HASH_SEED)
    h = h ^ (h >> jnp.uint32(16))
    s = (h % jnp.uint32(_SLOT_NUM)).astype(jnp.int32)
    slots = jax.lax.broadcasted_iota(jnp.int32, (block_b, _SLOT_NUM), 1)
    out_ref[0, :, :] = (slots == s).astype(jnp.float32)


def kernel(input_tensor):
    batch_size = input_tensor.shape[0]
    grid = (_DEP_DIM, batch_size // _BLOCK_B)
    return pl.pallas_call(
        functools.partial(_onehot_kernel, batch_size=batch_size,
                          block_b=_BLOCK_B),
        grid=grid,
        out_specs=pl.BlockSpec((1, _BLOCK_B, _SLOT_NUM), lambda d, i: (d, i, 0)),
        out_shape=jax.ShapeDtypeStruct((_DEP_DIM, batch_size, _SLOT_NUM),
                                       jnp.float32),
    )()
